# baseline (speedup 1.0000x reference)
import jax
import jax.numpy as jnp
from jax.experimental import pallas as pl
from jax.experimental.pallas import tpu as pltpu

_N = 5000
_T = 128
_NT = 40
_NPAD = _NT * _T
_K = 384
_IOU = 0.5
_MAXDET = 300
_KPAD = 304


def _kill(x1a, y1a, x2a, y2a, aa, x1b, y1b, x2b, y2b, ab):
    ltx = jnp.maximum(x1a, x1b)
    lty = jnp.maximum(y1a, y1b)
    rbx = jnp.minimum(x2a, x2b)
    rby = jnp.minimum(y2a, y2b)
    w = jnp.maximum(rbx - ltx, 0.0)
    h = jnp.maximum(rby - lty, 0.0)
    inter = w * h
    union = (aa + ab) - inter
    iou = inter / jnp.maximum(union, 1e-9)
    return jnp.where(iou > _IOU, 1.0, 0.0)


def _nms_small_body(d_ref, dt_ref, out_ref, cnt_ref):
    f32 = jnp.float32
    ii = jax.lax.broadcasted_iota(jnp.int32, (_K, _K), 0)
    jj = jax.lax.broadcasted_iota(jnp.int32, (_K, _K), 1)
    lower = jnp.where(jj < ii, 1.0, 0.0).astype(f32)
    kio = jax.lax.broadcasted_iota(jnp.int32, (1, _KPAD), 1).astype(f32)

    x1i = d_ref[:, 0:1]
    y1i = d_ref[:, 1:2]
    x2i = d_ref[:, 2:3]
    y2i = d_ref[:, 3:4]
    ai = (x2i - x1i) * (y2i - y1i)
    x1j = dt_ref[0:1, :]
    y1j = dt_ref[1:2, :]
    x2j = dt_ref[2:3, :]
    y2j = dt_ref[3:4, :]
    aj = (x2j - x1j) * (y2j - y1j)
    s_mat = _kill(x1i, y1i, x2i, y2i, ai, x1j, y1j, x2j, y2j, aj) * lower

    def fix_cond(fcarry):
        return jnp.logical_not(fcarry[1])

    def fix_body(fcarry):
        alive, _ = fcarry
        fcnt = jnp.dot(s_mat, alive, preferred_element_type=f32)
        new = jnp.where(fcnt < 0.5, 1.0, 0.0)
        return new, jnp.all(new == alive)

    alive, _ = jax.lax.while_loop(
        fix_cond, fix_body, (jnp.ones((_K, 1), f32), jnp.array(False)))

    rank = jnp.dot(lower, alive, preferred_element_type=f32)
    match = jnp.where(rank == kio, 1.0, 0.0) * alive
    dsel = d_ref[:, 4:12]
    out_ref[:, :] = jax.lax.dot_general(
        match, dsel, (((0,), (0,)), ((), ())),
        preferred_element_type=f32, precision=jax.lax.Precision.HIGHEST)
    cnt_ref[:, :] = jnp.full((1, 1), jnp.sum(alive), f32)


def _resolve_body(outk_ref, cnt_ref, b_ref, s_ref, out_ref, sw_ref):
    f32 = jnp.float32
    enough = cnt_ref[0, 0] >= float(_MAXDET)

    @pl.when(enough)
    def _copy():
        out_ref[:, :] = outk_ref[:, :]

    @pl.when(jnp.logical_not(enough))
    def _rerun():
        out_ref[:, :] = jnp.zeros((_KPAD, 8), f32)
        sw_ref[:, :] = s_ref[:, :]
        o1 = b_ref[4, :, :]
        o2 = b_ref[5, :, :]
        o3 = b_ref[6, :, :]
        o4 = b_ref[7, :, :]
        areap = (o3 - o1) * (o4 - o2)
        iot = (jax.lax.broadcasted_iota(jnp.int32, (_NT, _T), 0) * _T
               + jax.lax.broadcasted_iota(jnp.int32, (_NT, _T), 1)).astype(f32)

        def cond(carry):
            kept, mx = carry
            return jnp.logical_and(kept < _MAXDET, mx > -3e38)

        def body(carry):
            kept, mx = carry
            pick = jnp.min(jnp.where(sw_ref[:, :] == mx, iot, 3e9))
            pm = iot == pick
            sx1 = jnp.sum(jnp.where(pm, o1, 0.0))
            sy1 = jnp.sum(jnp.where(pm, o2, 0.0))
            sx2 = jnp.sum(jnp.where(pm, o3, 0.0))
            sy2 = jnp.sum(jnp.where(pm, o4, 0.0))
            sarea = (sx2 - sx1) * (sy2 - sy1)
            kv = _kill(o1, o2, o3, o4, areap, sx1, sy1, sx2, sy2, sarea)
            sw_ref[:, :] = jnp.where(kv > 0.0, -jnp.inf, sw_ref[:, :])
            for c in range(4):
                val = jnp.sum(jnp.where(pm, b_ref[c, :, :], 0.0))
                out_ref[pl.ds(kept, 1), c:c + 1] = jnp.full((1, 1), val, f32)
            out_ref[pl.ds(kept, 1), 4:5] = jnp.full((1, 1), mx, f32)
            return kept + 1, jnp.max(sw_ref[:, :])

        jax.lax.while_loop(
            cond, body, (jnp.zeros((), jnp.int32), jnp.max(sw_ref[:, :])))


def kernel(boxes, scores, labels):
    f32 = jnp.float32
    ls = labels.astype(f32)
    max_coord = jnp.max(boxes) + 1.0
    off = ls * max_coord

    vals, idx = jax.lax.top_k(scores, _K)
    bk = boxes[idx]
    dk = jnp.concatenate([
        bk + off[idx][:, None],
        bk,
        vals[:, None],
        jnp.zeros((_K, 7), f32),
    ], axis=1)
    out_k, cnt_k = pl.pallas_call(
        _nms_small_body,
        out_shape=(jax.ShapeDtypeStruct((_KPAD, 8), f32),
                   jax.ShapeDtypeStruct((1, 1), f32)),
    )(dk, dk.T)

    bt = boxes.T
    planes = jnp.full((9, _NPAD), -jnp.inf, f32)
    planes = planes.at[:4, :_N].set(bt)
    planes = planes.at[4:8, :_N].set(bt + off[None, :])
    planes = planes.at[:8, _N:].set(0.0)
    planes = planes.at[8, :_N].set(scores)
    planes = planes.reshape(9, _NT, _T)
    out = pl.pallas_call(
        _resolve_body,
        out_shape=jax.ShapeDtypeStruct((_KPAD, 8), f32),
        scratch_shapes=[pltpu.VMEM((_NT, _T), f32)],
    )(out_k, cnt_k, planes[:8], planes[8])
    return out[:_MAXDET, :5]


# speedup vs baseline: 1.0046x; 1.0046x over previous
import jax
import jax.numpy as jnp
from jax.experimental import pallas as pl
from jax.experimental.pallas import tpu as pltpu

_N = 5000
_T = 128
_NT = 40
_NPAD = _NT * _T
_K = 384
_IOU = 0.5
_MAXDET = 300
_KPAD = 304


def _kill(x1a, y1a, x2a, y2a, aa, x1b, y1b, x2b, y2b, ab):
    ltx = jnp.maximum(x1a, x1b)
    lty = jnp.maximum(y1a, y1b)
    rbx = jnp.minimum(x2a, x2b)
    rby = jnp.minimum(y2a, y2b)
    w = jnp.maximum(rbx - ltx, 0.0)
    h = jnp.maximum(rby - lty, 0.0)
    inter = w * h
    union = (aa + ab) - inter
    iou = inter / jnp.maximum(union, 1e-9)
    return jnp.where(iou > _IOU, 1.0, 0.0)


def _nms_small_body(d_ref, dt_ref, out_ref, cnt_ref):
    f32 = jnp.float32
    ii = jax.lax.broadcasted_iota(jnp.int32, (_K, _K), 0)
    jj = jax.lax.broadcasted_iota(jnp.int32, (_K, _K), 1)
    lower = jnp.where(jj < ii, 1.0, 0.0).astype(f32)
    kio = jax.lax.broadcasted_iota(jnp.int32, (1, _KPAD), 1).astype(f32)

    x1i = d_ref[:, 0:1]
    y1i = d_ref[:, 1:2]
    x2i = d_ref[:, 2:3]
    y2i = d_ref[:, 3:4]
    ai = (x2i - x1i) * (y2i - y1i)
    x1j = dt_ref[0:1, :]
    y1j = dt_ref[1:2, :]
    x2j = dt_ref[2:3, :]
    y2j = dt_ref[3:4, :]
    aj = (x2j - x1j) * (y2j - y1j)
    s_mat = _kill(x1i, y1i, x2i, y2i, ai, x1j, y1j, x2j, y2j, aj) * lower

    def fix_cond(fcarry):
        return jnp.logical_not(fcarry[1])

    def fix_body(fcarry):
        alive, _ = fcarry
        fcnt = jnp.dot(s_mat, alive, preferred_element_type=f32)
        new = jnp.where(fcnt < 0.5, 1.0, 0.0)
        return new, jnp.all(new == alive)

    alive, _ = jax.lax.while_loop(
        fix_cond, fix_body, (jnp.ones((_K, 1), f32), jnp.array(False)))

    rank = jnp.dot(lower, alive, preferred_element_type=f32)
    match = jnp.where(rank == kio, 1.0, 0.0) * alive
    dsel = d_ref[:, 4:12]
    out_ref[:, :] = jax.lax.dot_general(
        match, dsel, (((0,), (0,)), ((), ())),
        preferred_element_type=f32, precision=jax.lax.Precision.HIGHEST)
    cnt_ref[:, :] = jnp.full((1, 1), jnp.sum(alive), f32)


def _resolve_body(outk_ref, cnt_ref, b_ref, s_ref, out_ref, bvm_ref, sw_ref,
                  sem_ref, sem2_ref):
    f32 = jnp.float32
    enough = cnt_ref[0, 0] >= float(_MAXDET)

    @pl.when(enough)
    def _copy():
        out_ref[:, :] = outk_ref[:, :]

    @pl.when(jnp.logical_not(enough))
    def _rerun():
        cpb = pltpu.make_async_copy(b_ref, bvm_ref, sem_ref)
        cpb.start()
        cps = pltpu.make_async_copy(s_ref, sw_ref, sem2_ref)
        cps.start()
        out_ref[:, :] = jnp.zeros((_KPAD, 8), f32)
        cpb.wait()
        cps.wait()
        o1 = bvm_ref[4, :, :]
        o2 = bvm_ref[5, :, :]
        o3 = bvm_ref[6, :, :]
        o4 = bvm_ref[7, :, :]
        areap = (o3 - o1) * (o4 - o2)
        iot = (jax.lax.broadcasted_iota(jnp.int32, (_NT, _T), 0) * _T
               + jax.lax.broadcasted_iota(jnp.int32, (_NT, _T), 1)).astype(f32)

        def cond(carry):
            kept, mx = carry
            return jnp.logical_and(kept < _MAXDET, mx > -3e38)

        def body(carry):
            kept, mx = carry
            pick = jnp.min(jnp.where(sw_ref[:, :] == mx, iot, 3e9))
            pm = iot == pick
            sx1 = jnp.sum(jnp.where(pm, o1, 0.0))
            sy1 = jnp.sum(jnp.where(pm, o2, 0.0))
            sx2 = jnp.sum(jnp.where(pm, o3, 0.0))
            sy2 = jnp.sum(jnp.where(pm, o4, 0.0))
            sarea = (sx2 - sx1) * (sy2 - sy1)
            kv = _kill(o1, o2, o3, o4, areap, sx1, sy1, sx2, sy2, sarea)
            sw_ref[:, :] = jnp.where(kv > 0.0, -jnp.inf, sw_ref[:, :])
            for c in range(4):
                val = jnp.sum(jnp.where(pm, bvm_ref[c, :, :], 0.0))
                out_ref[pl.ds(kept, 1), c:c + 1] = jnp.full((1, 1), val, f32)
            out_ref[pl.ds(kept, 1), 4:5] = jnp.full((1, 1), mx, f32)
            return kept + 1, jnp.max(sw_ref[:, :])

        jax.lax.while_loop(
            cond, body, (jnp.zeros((), jnp.int32), jnp.max(sw_ref[:, :])))


def kernel(boxes, scores, labels):
    f32 = jnp.float32
    ls = labels.astype(f32)
    max_coord = jnp.max(boxes) + 1.0
    off = ls * max_coord

    vals, idx = jax.lax.top_k(scores, _K)
    bk = boxes[idx]
    dk = jnp.concatenate([
        bk + off[idx][:, None],
        bk,
        vals[:, None],
        jnp.zeros((_K, 7), f32),
    ], axis=1)
    out_k, cnt_k = pl.pallas_call(
        _nms_small_body,
        out_shape=(jax.ShapeDtypeStruct((_KPAD, 8), f32),
                   jax.ShapeDtypeStruct((1, 1), f32)),
    )(dk, dk.T)

    bt = boxes.T
    planes = jnp.full((9, _NPAD), -jnp.inf, f32)
    planes = planes.at[:4, :_N].set(bt)
    planes = planes.at[4:8, :_N].set(bt + off[None, :])
    planes = planes.at[:8, _N:].set(0.0)
    planes = planes.at[8, :_N].set(scores)
    planes = planes.reshape(9, _NT, _T)
    out = pl.pallas_call(
        _resolve_body,
        out_shape=jax.ShapeDtypeStruct((_KPAD, 8), f32),
        in_specs=[pl.BlockSpec(memory_space=pltpu.VMEM),
                  pl.BlockSpec(memory_space=pltpu.VMEM),
                  pl.BlockSpec(memory_space=pltpu.MemorySpace.HBM),
                  pl.BlockSpec(memory_space=pltpu.MemorySpace.HBM)],
        scratch_shapes=[pltpu.VMEM((8, _NT, _T), f32),
                        pltpu.VMEM((_NT, _T), f32),
                        pltpu.SemaphoreType.DMA,
                        pltpu.SemaphoreType.DMA],
    )(out_k, cnt_k, planes[:8], planes[8])
    return out[:_MAXDET, :5]


# speedup vs baseline: 1.0841x; 1.0792x over previous
import jax
import jax.numpy as jnp
from jax.experimental import pallas as pl
from jax.experimental.pallas import tpu as pltpu

_N = 5000
_T = 128
_NT = 40
_NPAD = _NT * _T
_K = 384
_IOU = 0.5
_MAXDET = 300
_KPAD = 304


def _kill(x1a, y1a, x2a, y2a, aa, x1b, y1b, x2b, y2b, ab):
    ltx = jnp.maximum(x1a, x1b)
    lty = jnp.maximum(y1a, y1b)
    rbx = jnp.minimum(x2a, x2b)
    rby = jnp.minimum(y2a, y2b)
    w = jnp.maximum(rbx - ltx, 0.0)
    h = jnp.maximum(rby - lty, 0.0)
    inter = w * h
    union = (aa + ab) - inter
    iou = inter / jnp.maximum(union, 1e-9)
    return jnp.where(iou > _IOU, 1.0, 0.0)


def _nms_body(d_ref, dt_ref, b_ref, s_ref, out_ref, bvm_ref, sw_ref,
              sem_ref, sem2_ref):
    f32 = jnp.float32
    ii = jax.lax.broadcasted_iota(jnp.int32, (_K, _K), 0)
    jj = jax.lax.broadcasted_iota(jnp.int32, (_K, _K), 1)
    lower = jnp.where(jj < ii, 1.0, 0.0).astype(f32)
    kio = jax.lax.broadcasted_iota(jnp.int32, (1, _KPAD), 1).astype(f32)

    x1i = d_ref[:, 0:1]
    y1i = d_ref[:, 1:2]
    x2i = d_ref[:, 2:3]
    y2i = d_ref[:, 3:4]
    ai = (x2i - x1i) * (y2i - y1i)
    x1j = dt_ref[0:1, :]
    y1j = dt_ref[1:2, :]
    x2j = dt_ref[2:3, :]
    y2j = dt_ref[3:4, :]
    aj = (x2j - x1j) * (y2j - y1j)
    s_mat = _kill(x1i, y1i, x2i, y2i, ai, x1j, y1j, x2j, y2j, aj) * lower

    def fix_cond(fcarry):
        return jnp.logical_not(fcarry[1])

    def fix_body(fcarry):
        alive, _ = fcarry
        fcnt = jnp.dot(s_mat, alive, preferred_element_type=f32)
        new = jnp.where(fcnt < 0.5, 1.0, 0.0)
        return new, jnp.all(new == alive)

    alive, _ = jax.lax.while_loop(
        fix_cond, fix_body, (jnp.ones((_K, 1), f32), jnp.array(False)))

    rank = jnp.dot(lower, alive, preferred_element_type=f32)
    match = jnp.where(rank == kio, 1.0, 0.0) * alive
    dsel = d_ref[:, 4:12]
    out_ref[:, :] = jax.lax.dot_general(
        match, dsel, (((0,), (0,)), ((), ())),
        preferred_element_type=f32, precision=jax.lax.Precision.HIGHEST)

    enough = jnp.sum(alive) >= float(_MAXDET)

    @pl.when(jnp.logical_not(enough))
    def _rerun():
        cpb = pltpu.make_async_copy(b_ref, bvm_ref, sem_ref)
        cpb.start()
        cps = pltpu.make_async_copy(s_ref, sw_ref, sem2_ref)
        cps.start()
        out_ref[:, :] = jnp.zeros((_KPAD, 8), f32)
        cpb.wait()
        cps.wait()
        o1 = bvm_ref[4, :, :]
        o2 = bvm_ref[5, :, :]
        o3 = bvm_ref[6, :, :]
        o4 = bvm_ref[7, :, :]
        areap = (o3 - o1) * (o4 - o2)
        iot = (jax.lax.broadcasted_iota(jnp.int32, (_NT, _T), 0) * _T
               + jax.lax.broadcasted_iota(jnp.int32, (_NT, _T), 1)).astype(f32)

        def cond(carry):
            kept, mx = carry
            return jnp.logical_and(kept < _MAXDET, mx > -3e38)

        def body(carry):
            kept, mx = carry
            pick = jnp.min(jnp.where(sw_ref[:, :] == mx, iot, 3e9))
            pm = iot == pick
            sx1 = jnp.sum(jnp.where(pm, o1, 0.0))
            sy1 = jnp.sum(jnp.where(pm, o2, 0.0))
            sx2 = jnp.sum(jnp.where(pm, o3, 0.0))
            sy2 = jnp.sum(jnp.where(pm, o4, 0.0))
            sarea = (sx2 - sx1) * (sy2 - sy1)
            kv = _kill(o1, o2, o3, o4, areap, sx1, sy1, sx2, sy2, sarea)
            sw_ref[:, :] = jnp.where(kv > 0.0, -jnp.inf, sw_ref[:, :])
            for c in range(4):
                val = jnp.sum(jnp.where(pm, bvm_ref[c, :, :], 0.0))
                out_ref[pl.ds(kept, 1), c:c + 1] = jnp.full((1, 1), val, f32)
            out_ref[pl.ds(kept, 1), 4:5] = jnp.full((1, 1), mx, f32)
            return kept + 1, jnp.max(sw_ref[:, :])

        jax.lax.while_loop(
            cond, body, (jnp.zeros((), jnp.int32), jnp.max(sw_ref[:, :])))


def kernel(boxes, scores, labels):
    f32 = jnp.float32
    ls = labels.astype(f32)
    max_coord = jnp.max(boxes) + 1.0
    off = ls * max_coord

    vals, idx = jax.lax.top_k(scores, _K)
    bk = boxes[idx]
    dk = jnp.concatenate([
        bk + off[idx][:, None],
        bk,
        vals[:, None],
        jnp.zeros((_K, 7), f32),
    ], axis=1)
    bt = boxes.T
    planes = jnp.full((9, _NPAD), -jnp.inf, f32)
    planes = planes.at[:4, :_N].set(bt)
    planes = planes.at[4:8, :_N].set(bt + off[None, :])
    planes = planes.at[:8, _N:].set(0.0)
    planes = planes.at[8, :_N].set(scores)
    planes = planes.reshape(9, _NT, _T)

    out = pl.pallas_call(
        _nms_body,
        out_shape=jax.ShapeDtypeStruct((_KPAD, 8), f32),
        in_specs=[pl.BlockSpec(memory_space=pltpu.VMEM),
                  pl.BlockSpec(memory_space=pltpu.VMEM),
                  pl.BlockSpec(memory_space=pltpu.MemorySpace.HBM),
                  pl.BlockSpec(memory_space=pltpu.MemorySpace.HBM)],
        scratch_shapes=[pltpu.VMEM((8, _NT, _T), f32),
                        pltpu.VMEM((_NT, _T), f32),
                        pltpu.SemaphoreType.DMA,
                        pltpu.SemaphoreType.DMA],
    )(dk, dk.T, planes[:8], planes[8])
    return out[:_MAXDET, :5]
